# Initial kernel scaffold; baseline (speedup 1.0000x reference)
import jax
import jax.numpy as jnp
from jax.experimental import pallas as pl


def kernel(x, edge_index, W1, b1, W2, b2):
    raise NotImplementedError("write your pallas kernel here")


# speedup vs baseline: 1.9857x; 1.9857x over previous
import jax
import jax.numpy as jnp
from jax.experimental import pallas as pl

_N = 10000


def _dense_kernel(x_ref, w_ref, o_ref):
    o_ref[...] = jnp.dot(x_ref[...], w_ref[...], preferred_element_type=jnp.float32)


def _mm(a, b):
    return pl.pallas_call(
        _dense_kernel,
        out_shape=jax.ShapeDtypeStruct((a.shape[0], b.shape[1]), jnp.float32),
    )(a, b)


def kernel(x, edge_index, W1, b1, W2, b2):
    loops = jnp.arange(_N, dtype=edge_index.dtype)
    src = jnp.concatenate([edge_index[0], loops])
    dst = jnp.concatenate([edge_index[1], loops])
    deg = jax.ops.segment_sum(jnp.ones_like(src, jnp.float32), dst, num_segments=_N)
    dinv = jnp.where(deg > 0, jax.lax.rsqrt(jnp.maximum(deg, 1e-12)), 0.0)

    g1 = dinv[:, None] * _mm(x, W1)
    agg1 = jax.ops.segment_sum(g1[src], dst, num_segments=_N)
    z1 = jax.nn.relu(dinv[:, None] * agg1 + b1)

    g2 = dinv[:, None] * _mm(z1, W2)
    agg2 = jax.ops.segment_sum(g2[src], dst, num_segments=_N)
    out = dinv[:, None] * agg2 + b2
    return jax.nn.log_softmax(out, axis=1)


# speedup vs baseline: 14.8731x; 7.4903x over previous
import functools

import jax
import jax.numpy as jnp
from jax import lax
from jax.experimental import pallas as pl
from jax.experimental.pallas import tpu as pltpu
from jax.experimental.pallas import tpu_sc as plsc

_N = 10000
_NCLS = 40
_D1 = 128
_D2 = 48
_NC = 2
_NS = 16
_NW = _NC * _NS
_C = 128
_NCH = 82
_EPAD = _NW * _NCH * _C
_NPAD = 10240
_RPT = _NPAD // _NS
_DEGW = 16


def _sc_agg(g, srcs, dsts, zrows, d):
    mesh = plsc.VectorSubcoreMesh(core_axis_name="c", subcore_axis_name="s")

    @functools.partial(
        pl.kernel,
        out_type=jax.ShapeDtypeStruct((_NC, _NPAD, d), jnp.float32),
        mesh=mesh,
        scratch_types=[
            pltpu.VMEM((2, _C), jnp.int32),
            pltpu.VMEM((2, _C), jnp.int32),
            pltpu.VMEM((2, _C, d), jnp.float32),
            pltpu.VMEM_SHARED((_NPAD, d), jnp.float32),
            pltpu.SemaphoreType.DMA,
            pltpu.SemaphoreType.DMA,
        ],
        compiler_params=pltpu.CompilerParams(use_tc_tiling_on_sc=False),
    )
    def k(g_hbm, src_hbm, dst_hbm, z_hbm, out_hbm,
          srcix, dstix, buf_v, acc_sh, isem, gsem):
        cid = lax.axis_index("c")
        sid = lax.axis_index("s")
        wid = sid * _NC + cid
        pltpu.sync_copy(z_hbm, buf_v.at[0])
        for r in range(_RPT // _C):
            pltpu.sync_copy(buf_v.at[0], acc_sh.at[pl.ds(sid * _RPT + r * _C, _C)])
        plsc.subcore_barrier()
        pltpu.async_copy(src_hbm.at[wid, 0], srcix.at[0], isem)
        pltpu.async_copy(dst_hbm.at[wid, 0], dstix.at[0], isem)
        pltpu.async_copy(src_hbm.at[wid, 1], srcix.at[1], isem)
        pltpu.async_copy(dst_hbm.at[wid, 1], dstix.at[1], isem)
        pltpu.make_async_copy(src_hbm.at[wid, 0], srcix.at[0], isem).wait()
        pltpu.make_async_copy(dst_hbm.at[wid, 0], dstix.at[0], isem).wait()
        pltpu.async_copy(g_hbm.at[srcix.at[0]], buf_v.at[0], gsem)

        def pair(o, carry):
            for b in range(2):
                j = o * 2 + b
                other = 1 - b

                @pl.when(j + 1 < _NCH)
                def _():
                    pltpu.make_async_copy(
                        src_hbm.at[wid, j + 1], srcix.at[other], isem).wait()
                    pltpu.make_async_copy(
                        dst_hbm.at[wid, j + 1], dstix.at[other], isem).wait()
                    pltpu.async_copy(g_hbm.at[srcix.at[other]], buf_v.at[other], gsem)

                pltpu.make_async_copy(g_hbm.at[srcix.at[b]], buf_v.at[b], gsem).wait()
                pltpu.sync_copy(buf_v.at[b], acc_sh.at[dstix.at[b]], add=True)

                @pl.when(j + 2 < _NCH)
                def _():
                    pltpu.async_copy(src_hbm.at[wid, j + 2], srcix.at[b], isem)
                    pltpu.async_copy(dst_hbm.at[wid, j + 2], dstix.at[b], isem)

            return carry

        lax.fori_loop(0, _NCH // 2, pair, 0)
        plsc.subcore_barrier()
        pltpu.sync_copy(acc_sh.at[pl.ds(sid * _RPT, _RPT)],
                        out_hbm.at[cid, pl.ds(sid * _RPT, _RPT)])

    return k(g, srcs, dsts, zrows)


def _sc_degree(dsts, zo):
    mesh = plsc.VectorSubcoreMesh(core_axis_name="c", subcore_axis_name="s")

    @functools.partial(
        pl.kernel,
        out_type=jax.ShapeDtypeStruct((_NC, _NPAD, _DEGW), jnp.float32),
        mesh=mesh,
        scratch_types=[
            pltpu.VMEM((_NCH, _C), jnp.int32),
            pltpu.VMEM((_C, _DEGW), jnp.float32),
            pltpu.VMEM_SHARED((_NPAD, _DEGW), jnp.float32),
        ],
        compiler_params=pltpu.CompilerParams(use_tc_tiling_on_sc=False),
    )
    def k(dst_hbm, zo_hbm, out_hbm, dst_v, val_v, acc_sh):
        cid = lax.axis_index("c")
        sid = lax.axis_index("s")
        wid = sid * _NC + cid
        pltpu.sync_copy(dst_hbm.at[wid], dst_v)
        pltpu.sync_copy(zo_hbm.at[0], val_v)
        for r in range(_RPT // _C):
            pltpu.sync_copy(val_v, acc_sh.at[pl.ds(sid * _RPT + r * _C, _C)])
        plsc.subcore_barrier()
        pltpu.sync_copy(zo_hbm.at[1], val_v)

        def body(j, carry):
            pltpu.sync_copy(val_v, acc_sh.at[dst_v.at[j]], add=True)
            return carry

        lax.fori_loop(0, _NCH, body, 0)
        plsc.subcore_barrier()
        pltpu.sync_copy(acc_sh.at[pl.ds(sid * _RPT, _RPT)],
                        out_hbm.at[cid, pl.ds(sid * _RPT, _RPT)])

    return k(dsts, zo)


def _dinv(degp_ref):
    deg = degp_ref[0] + degp_ref[1]
    return jnp.where(deg > 0, lax.rsqrt(jnp.maximum(deg, 1e-12)), 0.0)


def _tc1_kernel(degp_ref, x_ref, w1_ref, g_ref):
    dinv = _dinv(degp_ref)
    h = jnp.dot(x_ref[...], w1_ref[...], preferred_element_type=jnp.float32)
    g_ref[...] = dinv[:_N, 0:1] * h


def _tc2_kernel(p1_ref, degp_ref, b1_ref, w2_ref, g2_ref):
    dinv = _dinv(degp_ref)
    agg = p1_ref[0, :_N] + p1_ref[1, :_N]
    z = jnp.maximum(dinv[:_N, 0:1] * agg + b1_ref[...], 0.0)
    h2 = jnp.dot(z, w2_ref[...], preferred_element_type=jnp.float32)
    g2_ref[...] = dinv[:_N, 0:1] * h2


def _tc3_kernel(p2_ref, degp_ref, b2_ref, o_ref):
    dinv = _dinv(degp_ref)
    agg = p2_ref[0, :_N] + p2_ref[1, :_N]
    o = dinv[:_N, 0:1] * agg + b2_ref[...]
    col = lax.broadcasted_iota(jnp.int32, (_N, _D2), 1)
    logits = jnp.where(col < _NCLS, o, -jnp.inf)
    m = jnp.max(logits, axis=1, keepdims=True)
    ex = jnp.where(col < _NCLS, jnp.exp(o - m), 0.0)
    lse = jnp.log(jnp.sum(ex, axis=1, keepdims=True))
    o_ref[...] = o - m - lse


def kernel(x, edge_index, W1, b1, W2, b2):
    loops = jnp.arange(_N, dtype=jnp.int32)
    src = jnp.concatenate([edge_index[0].astype(jnp.int32), loops])
    dst = jnp.concatenate([edge_index[1].astype(jnp.int32), loops])
    pad_n = _EPAD - src.shape[0]
    srcs = jnp.concatenate([src, jnp.zeros((pad_n,), jnp.int32)]).reshape(_NW, _NCH, _C)
    dsts = jnp.concatenate(
        [dst, jnp.full((pad_n,), _NPAD - 1, jnp.int32)]).reshape(_NW, _NCH, _C)

    zo = jnp.zeros((2, _C, _DEGW), jnp.float32).at[1, :, 0].set(1.0)
    z1 = jnp.zeros((_C, _D1), jnp.float32)
    z2 = jnp.zeros((_C, _D2), jnp.float32)
    w2p = jnp.pad(W2, ((0, 0), (0, _D2 - _NCLS)))
    b2p = jnp.pad(b2, (0, _D2 - _NCLS))

    degp = _sc_degree(dsts, zo)

    g1 = pl.pallas_call(
        _tc1_kernel,
        out_shape=jax.ShapeDtypeStruct((_N, _D1), jnp.float32),
    )(degp, x, W1)

    p1 = _sc_agg(g1, srcs, dsts, z1, _D1)

    g2 = pl.pallas_call(
        _tc2_kernel,
        out_shape=jax.ShapeDtypeStruct((_N, _D2), jnp.float32),
    )(p1, degp, b1, w2p)

    p2 = _sc_agg(g2, srcs, dsts, z2, _D2)

    out = pl.pallas_call(
        _tc3_kernel,
        out_shape=jax.ShapeDtypeStruct((_N, _D2), jnp.float32),
    )(p2, degp, b2p)

    return out[:, :_NCLS]


# speedup vs baseline: 15.2055x; 1.0223x over previous
import functools

import jax
import jax.numpy as jnp
from jax import lax
from jax.experimental import pallas as pl
from jax.experimental.pallas import tpu as pltpu
from jax.experimental.pallas import tpu_sc as plsc

_N = 10000
_NCLS = 40
_D1 = 128
_D2 = 48
_NC = 2
_NS = 16
_NW = _NC * _NS
_C = 128
_NCH = 82
_EPAD = _NW * _NCH * _C
_NPAD = 10240
_RPT = _NPAD // _NS
_DEGW = 16


def _sc_agg(g, srcs, dsts, zrows, d):
    mesh = plsc.VectorSubcoreMesh(core_axis_name="c", subcore_axis_name="s")

    @functools.partial(
        pl.kernel,
        out_type=jax.ShapeDtypeStruct((_NC, _NPAD, d), jnp.float32),
        mesh=mesh,
        scratch_types=[
            pltpu.VMEM((4, _C), jnp.int32),
            pltpu.VMEM((4, _C), jnp.int32),
            pltpu.VMEM((2, _C, d), jnp.float32),
            pltpu.VMEM_SHARED((_NPAD, d), jnp.float32),
            pltpu.SemaphoreType.DMA,
            pltpu.SemaphoreType.DMA,
            pltpu.SemaphoreType.DMA,
        ],
        compiler_params=pltpu.CompilerParams(use_tc_tiling_on_sc=False),
    )
    def k(g_hbm, src_hbm, dst_hbm, z_hbm, out_hbm,
          srcix, dstix, buf_v, acc_sh, isem, gsem, ssem):
        cid = lax.axis_index("c")
        sid = lax.axis_index("s")
        wid = sid * _NC + cid
        pltpu.sync_copy(z_hbm, buf_v.at[0])
        for r in range(_RPT // _C):
            pltpu.sync_copy(buf_v.at[0], acc_sh.at[pl.ds(sid * _RPT + r * _C, _C)])
        plsc.subcore_barrier()
        pltpu.async_copy(src_hbm.at[wid, 0], srcix.at[0], isem)
        pltpu.async_copy(dst_hbm.at[wid, 0], dstix.at[0], isem)
        pltpu.async_copy(src_hbm.at[wid, 1], srcix.at[1], isem)
        pltpu.async_copy(dst_hbm.at[wid, 1], dstix.at[1], isem)
        pltpu.make_async_copy(src_hbm.at[wid, 0], srcix.at[0], isem).wait()
        pltpu.make_async_copy(dst_hbm.at[wid, 0], dstix.at[0], isem).wait()
        pltpu.async_copy(g_hbm.at[srcix.at[0]], buf_v.at[0], gsem)

        def pair(o, carry):
            for b in range(2):
                j = o * 2 + b
                bs = (o * 2 + b) % 4
                other = 1 - b
                nbs = (bs + 1) % 4
                pbs = (bs + 3) % 4

                @pl.when(j + 1 < _NCH)
                def _():
                    pltpu.make_async_copy(
                        src_hbm.at[wid, j + 1], srcix.at[nbs], isem).wait()
                    pltpu.make_async_copy(
                        dst_hbm.at[wid, j + 1], dstix.at[nbs], isem).wait()

                    @pl.when(j >= 1)
                    def _():
                        pltpu.make_async_copy(
                            buf_v.at[other], acc_sh.at[dstix.at[pbs]], ssem).wait()

                    pltpu.async_copy(g_hbm.at[srcix.at[nbs]], buf_v.at[other], gsem)

                pltpu.make_async_copy(g_hbm.at[srcix.at[bs]], buf_v.at[b], gsem).wait()
                pltpu.async_copy(buf_v.at[b], acc_sh.at[dstix.at[bs]], ssem, add=True)

                @pl.when(j + 2 < _NCH)
                def _():
                    pltpu.async_copy(src_hbm.at[wid, j + 2], srcix.at[(bs + 2) % 4], isem)
                    pltpu.async_copy(dst_hbm.at[wid, j + 2], dstix.at[(bs + 2) % 4], isem)

            return carry

        lax.fori_loop(0, _NCH // 2, pair, 0)
        pltpu.make_async_copy(
            buf_v.at[0], acc_sh.at[dstix.at[(_NCH - 2) % 4]], ssem).wait()
        pltpu.make_async_copy(
            buf_v.at[1], acc_sh.at[dstix.at[(_NCH - 1) % 4]], ssem).wait()
        plsc.subcore_barrier()
        pltpu.sync_copy(acc_sh.at[pl.ds(sid * _RPT, _RPT)],
                        out_hbm.at[cid, pl.ds(sid * _RPT, _RPT)])

    return k(g, srcs, dsts, zrows)


def _sc_degree(dsts, zo):
    mesh = plsc.VectorSubcoreMesh(core_axis_name="c", subcore_axis_name="s")

    @functools.partial(
        pl.kernel,
        out_type=jax.ShapeDtypeStruct((_NC, _NPAD, _DEGW), jnp.float32),
        mesh=mesh,
        scratch_types=[
            pltpu.VMEM((_NCH, _C), jnp.int32),
            pltpu.VMEM((_C, _DEGW), jnp.float32),
            pltpu.VMEM_SHARED((_NPAD, _DEGW), jnp.float32),
        ],
        compiler_params=pltpu.CompilerParams(use_tc_tiling_on_sc=False),
    )
    def k(dst_hbm, zo_hbm, out_hbm, dst_v, val_v, acc_sh):
        cid = lax.axis_index("c")
        sid = lax.axis_index("s")
        wid = sid * _NC + cid
        pltpu.sync_copy(dst_hbm.at[wid], dst_v)
        pltpu.sync_copy(zo_hbm.at[0], val_v)
        for r in range(_RPT // _C):
            pltpu.sync_copy(val_v, acc_sh.at[pl.ds(sid * _RPT + r * _C, _C)])
        plsc.subcore_barrier()
        pltpu.sync_copy(zo_hbm.at[1], val_v)

        def body(j, carry):
            pltpu.sync_copy(val_v, acc_sh.at[dst_v.at[j]], add=True)
            return carry

        lax.fori_loop(0, _NCH, body, 0)
        plsc.subcore_barrier()
        pltpu.sync_copy(acc_sh.at[pl.ds(sid * _RPT, _RPT)],
                        out_hbm.at[cid, pl.ds(sid * _RPT, _RPT)])

    return k(dsts, zo)


def _dinv(degp_ref):
    deg = degp_ref[0] + degp_ref[1]
    return jnp.where(deg > 0, lax.rsqrt(jnp.maximum(deg, 1e-12)), 0.0)


def _tc1_kernel(degp_ref, x_ref, w1_ref, g_ref):
    dinv = _dinv(degp_ref)
    h = jnp.dot(x_ref[...], w1_ref[...], preferred_element_type=jnp.float32)
    g_ref[...] = dinv[:_N, 0:1] * h


def _tc2_kernel(p1_ref, degp_ref, b1_ref, w2_ref, g2_ref):
    dinv = _dinv(degp_ref)
    agg = p1_ref[0, :_N] + p1_ref[1, :_N]
    z = jnp.maximum(dinv[:_N, 0:1] * agg + b1_ref[...], 0.0)
    h2 = jnp.dot(z, w2_ref[...], preferred_element_type=jnp.float32)
    g2_ref[...] = dinv[:_N, 0:1] * h2


def _tc3_kernel(p2_ref, degp_ref, b2_ref, o_ref):
    dinv = _dinv(degp_ref)
    agg = p2_ref[0, :_N] + p2_ref[1, :_N]
    o = dinv[:_N, 0:1] * agg + b2_ref[...]
    col = lax.broadcasted_iota(jnp.int32, (_N, _D2), 1)
    logits = jnp.where(col < _NCLS, o, -jnp.inf)
    m = jnp.max(logits, axis=1, keepdims=True)
    ex = jnp.where(col < _NCLS, jnp.exp(o - m), 0.0)
    lse = jnp.log(jnp.sum(ex, axis=1, keepdims=True))
    o_ref[...] = o - m - lse


def kernel(x, edge_index, W1, b1, W2, b2):
    loops = jnp.arange(_N, dtype=jnp.int32)
    src = jnp.concatenate([edge_index[0].astype(jnp.int32), loops])
    dst = jnp.concatenate([edge_index[1].astype(jnp.int32), loops])
    pad_n = _EPAD - src.shape[0]
    srcs = jnp.concatenate([src, jnp.zeros((pad_n,), jnp.int32)]).reshape(_NW, _NCH, _C)
    dsts = jnp.concatenate(
        [dst, jnp.full((pad_n,), _NPAD - 1, jnp.int32)]).reshape(_NW, _NCH, _C)

    zo = jnp.zeros((2, _C, _DEGW), jnp.float32).at[1, :, 0].set(1.0)
    z1 = jnp.zeros((_C, _D1), jnp.float32)
    z2 = jnp.zeros((_C, _D2), jnp.float32)
    w2p = jnp.pad(W2, ((0, 0), (0, _D2 - _NCLS)))
    b2p = jnp.pad(b2, (0, _D2 - _NCLS))

    degp = _sc_degree(dsts, zo)

    g1 = pl.pallas_call(
        _tc1_kernel,
        out_shape=jax.ShapeDtypeStruct((_N, _D1), jnp.float32),
    )(degp, x, W1)

    p1 = _sc_agg(g1, srcs, dsts, z1, _D1)

    g2 = pl.pallas_call(
        _tc2_kernel,
        out_shape=jax.ShapeDtypeStruct((_N, _D2), jnp.float32),
    )(p1, degp, b1, w2p)

    p2 = _sc_agg(g2, srcs, dsts, z2, _D2)

    out = pl.pallas_call(
        _tc3_kernel,
        out_shape=jax.ShapeDtypeStruct((_N, _D2), jnp.float32),
    )(p2, degp, b2p)

    return out[:, :_NCLS]


# speedup vs baseline: 17.7948x; 1.1703x over previous
import functools

import jax
import jax.numpy as jnp
from jax import lax
from jax.experimental import pallas as pl
from jax.experimental.pallas import tpu as pltpu
from jax.experimental.pallas import tpu_sc as plsc

_N = 10000
_NCLS = 40
_D1 = 128
_D2 = 48
_NC = 2
_NS = 16
_NW = _NC * _NS
_NPAD = 10240
_RPT = _NPAD // _NS
_DEGW = 16
_C1, _NCH1 = 96, 108
_C2, _NCH2 = 128, 84


def _sc_agg(g, srcs, dsts, zrows, d, c, nch, nb, gd):
    ni = 2 * nb
    assert gd <= nb - 2 and nch % ni == 0
    mesh = plsc.VectorSubcoreMesh(core_axis_name="c", subcore_axis_name="s")

    @functools.partial(
        pl.kernel,
        out_type=jax.ShapeDtypeStruct((_NC, _NPAD, d), jnp.float32),
        mesh=mesh,
        scratch_types=[
            pltpu.VMEM((ni, c), jnp.int32),
            pltpu.VMEM((ni, c), jnp.int32),
            pltpu.VMEM((nb, c, d), jnp.float32),
            pltpu.VMEM_SHARED((_NPAD, d), jnp.float32),
            pltpu.SemaphoreType.DMA,
            pltpu.SemaphoreType.DMA,
            pltpu.SemaphoreType.DMA,
        ],
        compiler_params=pltpu.CompilerParams(use_tc_tiling_on_sc=False),
    )
    def k(g_hbm, src_hbm, dst_hbm, z_hbm, out_hbm,
          srcix, dstix, buf_v, acc_sh, isem, gsem, ssem):
        cid = lax.axis_index("c")
        sid = lax.axis_index("s")
        wid = sid * _NC + cid
        pltpu.sync_copy(z_hbm, buf_v.at[0])
        for r in range(_RPT // c):
            pltpu.sync_copy(buf_v.at[0], acc_sh.at[pl.ds(sid * _RPT + r * c, c)])
        if _RPT % c:
            pltpu.sync_copy(buf_v.at[0, pl.ds(0, _RPT % c)],
                            acc_sh.at[pl.ds(sid * _RPT + (_RPT // c) * c, _RPT % c)])
        plsc.subcore_barrier()

        def idx_issue(j, s):
            pltpu.async_copy(src_hbm.at[wid, j], srcix.at[s], isem)
            pltpu.async_copy(dst_hbm.at[wid, j], dstix.at[s], isem)

        def idx_wait(j, s):
            pltpu.make_async_copy(src_hbm.at[wid, j], srcix.at[s], isem).wait()
            pltpu.make_async_copy(dst_hbm.at[wid, j], dstix.at[s], isem).wait()

        def gat_issue(s, bs):
            pltpu.async_copy(g_hbm.at[srcix.at[s]], buf_v.at[bs], gsem)

        def gat_wait(s, bs):
            pltpu.make_async_copy(g_hbm.at[srcix.at[s]], buf_v.at[bs], gsem).wait()

        def sca_issue(s, bs):
            pltpu.async_copy(buf_v.at[bs], acc_sh.at[dstix.at[s]], ssem, add=True)

        def sca_wait(s, bs):
            pltpu.make_async_copy(buf_v.at[bs], acc_sh.at[dstix.at[s]], ssem).wait()

        for t in range(nb - 1):
            idx_issue(t, t)
        for t in range(gd):
            idx_wait(t, t)
            gat_issue(t, t)

        def group(o, carry):
            j0 = o * ni
            for u in range(ni):
                j = j0 + u

                @pl.when(j + nb - 1 < nch)
                def _():
                    idx_issue(j + nb - 1, (u + nb - 1) % ni)

                @pl.when(j + gd < nch)
                def _():
                    @pl.when(j + gd - nb >= 0)
                    def _():
                        sca_wait((u + gd + nb) % ni, (u + gd) % nb)

                    idx_wait(j + gd, (u + gd) % ni)
                    gat_issue((u + gd) % ni, (u + gd) % nb)

                gat_wait(u % ni, u % nb)
                sca_issue(u % ni, u % nb)

            return carry

        lax.fori_loop(0, nch // ni, group, 0)
        for t in range(nb):
            kk = nch - nb + t
            sca_wait(kk % ni, kk % nb)
        plsc.subcore_barrier()
        pltpu.sync_copy(acc_sh.at[pl.ds(sid * _RPT, _RPT)],
                        out_hbm.at[cid, pl.ds(sid * _RPT, _RPT)])

    return k(g, srcs, dsts, zrows)


def _sc_degree(dsts, zo):
    mesh = plsc.VectorSubcoreMesh(core_axis_name="c", subcore_axis_name="s")

    @functools.partial(
        pl.kernel,
        out_type=jax.ShapeDtypeStruct((_NC, _NPAD, _DEGW), jnp.float32),
        mesh=mesh,
        scratch_types=[
            pltpu.VMEM((_NCH2, _C2), jnp.int32),
            pltpu.VMEM((_C2, _DEGW), jnp.float32),
            pltpu.VMEM_SHARED((_NPAD, _DEGW), jnp.float32),
        ],
        compiler_params=pltpu.CompilerParams(use_tc_tiling_on_sc=False),
    )
    def k(dst_hbm, zo_hbm, out_hbm, dst_v, val_v, acc_sh):
        cid = lax.axis_index("c")
        sid = lax.axis_index("s")
        wid = sid * _NC + cid
        pltpu.sync_copy(dst_hbm.at[wid], dst_v)
        pltpu.sync_copy(zo_hbm.at[0], val_v)
        for r in range(_RPT // _C2):
            pltpu.sync_copy(val_v, acc_sh.at[pl.ds(sid * _RPT + r * _C2, _C2)])
        plsc.subcore_barrier()
        pltpu.sync_copy(zo_hbm.at[1], val_v)

        def body(j, carry):
            pltpu.sync_copy(val_v, acc_sh.at[dst_v.at[j]], add=True)
            return carry

        lax.fori_loop(0, _NCH2, body, 0)
        plsc.subcore_barrier()
        pltpu.sync_copy(acc_sh.at[pl.ds(sid * _RPT, _RPT)],
                        out_hbm.at[cid, pl.ds(sid * _RPT, _RPT)])

    return k(dsts, zo)


def _dinv(degp_ref):
    deg = degp_ref[0] + degp_ref[1]
    return jnp.where(deg > 0, lax.rsqrt(jnp.maximum(deg, 1e-12)), 0.0)


def _tc1_kernel(degp_ref, x_ref, w1_ref, g_ref):
    dinv = _dinv(degp_ref)
    h = jnp.dot(x_ref[...], w1_ref[...], preferred_element_type=jnp.float32)
    g_ref[...] = dinv[:_N, 0:1] * h


def _tc2_kernel(p1_ref, degp_ref, b1_ref, w2_ref, g2_ref):
    dinv = _dinv(degp_ref)
    agg = p1_ref[0, :_N] + p1_ref[1, :_N]
    z = jnp.maximum(dinv[:_N, 0:1] * agg + b1_ref[...], 0.0)
    h2 = jnp.dot(z, w2_ref[...], preferred_element_type=jnp.float32)
    g2_ref[...] = dinv[:_N, 0:1] * h2


def _tc3_kernel(p2_ref, degp_ref, b2_ref, o_ref):
    dinv = _dinv(degp_ref)
    agg = p2_ref[0, :_N] + p2_ref[1, :_N]
    o = dinv[:_N, 0:1] * agg + b2_ref[...]
    col = lax.broadcasted_iota(jnp.int32, (_N, _D2), 1)
    logits = jnp.where(col < _NCLS, o, -jnp.inf)
    m = jnp.max(logits, axis=1, keepdims=True)
    ex = jnp.where(col < _NCLS, jnp.exp(o - m), 0.0)
    lse = jnp.log(jnp.sum(ex, axis=1, keepdims=True))
    o_ref[...] = o - m - lse


def _edge_layout(src, dst, c, nch):
    epad = _NW * nch * c
    pad_n = epad - src.shape[0]
    s = jnp.concatenate([src, jnp.zeros((pad_n,), jnp.int32)]).reshape(_NW, nch, c)
    t = jnp.concatenate(
        [dst, jnp.full((pad_n,), _NPAD - 1, jnp.int32)]).reshape(_NW, nch, c)
    return s, t


def kernel(x, edge_index, W1, b1, W2, b2):
    loops = jnp.arange(_N, dtype=jnp.int32)
    src = jnp.concatenate([edge_index[0].astype(jnp.int32), loops])
    dst = jnp.concatenate([edge_index[1].astype(jnp.int32), loops])
    srcs1, dsts1 = _edge_layout(src, dst, _C1, _NCH1)
    srcs2, dsts2 = _edge_layout(src, dst, _C2, _NCH2)

    zo = jnp.zeros((2, _C2, _DEGW), jnp.float32).at[1, :, 0].set(1.0)
    z1 = jnp.zeros((_C1, _D1), jnp.float32)
    z2 = jnp.zeros((_C2, _D2), jnp.float32)
    w2p = jnp.pad(W2, ((0, 0), (0, _D2 - _NCLS)))
    b2p = jnp.pad(b2, (0, _D2 - _NCLS))

    degp = _sc_degree(dsts2, zo)

    g1 = pl.pallas_call(
        _tc1_kernel,
        out_shape=jax.ShapeDtypeStruct((_N, _D1), jnp.float32),
    )(degp, x, W1)

    p1 = _sc_agg(g1, srcs1, dsts1, z1, _D1, _C1, _NCH1, 3, 1)

    g2 = pl.pallas_call(
        _tc2_kernel,
        out_shape=jax.ShapeDtypeStruct((_N, _D2), jnp.float32),
    )(p1, degp, b1, w2p)

    p2 = _sc_agg(g2, srcs2, dsts2, z2, _D2, _C2, _NCH2, 6, 2)

    out = pl.pallas_call(
        _tc3_kernel,
        out_shape=jax.ShapeDtypeStruct((_N, _D2), jnp.float32),
    )(p2, degp, b2p)

    return out[:, :_NCLS]


# speedup vs baseline: 17.8488x; 1.0030x over previous
import functools

import jax
import jax.numpy as jnp
from jax import lax
from jax.experimental import pallas as pl
from jax.experimental.pallas import tpu as pltpu
from jax.experimental.pallas import tpu_sc as plsc

_N = 10000
_NCLS = 40
_D1 = 128
_D2 = 48
_NC = 2
_NS = 16
_NW = _NC * _NS
_NPAD = 10240
_RPT = _NPAD // _NS
_DEGW = 16
_C1, _NCH1 = 96, 108
_C2, _NCH2 = 128, 84


def _sc_agg(g, srcs, dsts, zrows, d, c, nch, nb, gd):
    ni = 2 * nb
    assert gd <= nb - 2 and nch % ni == 0
    mesh = plsc.VectorSubcoreMesh(core_axis_name="c", subcore_axis_name="s")

    @functools.partial(
        pl.kernel,
        out_type=jax.ShapeDtypeStruct((_NC, _NPAD, d), jnp.float32),
        mesh=mesh,
        scratch_types=[
            pltpu.VMEM((ni, c), jnp.int32),
            pltpu.VMEM((ni, c), jnp.int32),
            pltpu.VMEM((nb, c, d), jnp.float32),
            pltpu.VMEM_SHARED((_NPAD, d), jnp.float32),
            pltpu.SemaphoreType.DMA,
            pltpu.SemaphoreType.DMA,
            pltpu.SemaphoreType.DMA,
        ],
        compiler_params=pltpu.CompilerParams(use_tc_tiling_on_sc=False),
    )
    def k(g_hbm, src_hbm, dst_hbm, z_hbm, out_hbm,
          srcix, dstix, buf_v, acc_sh, isem, gsem, ssem):
        cid = lax.axis_index("c")
        sid = lax.axis_index("s")
        wid = sid * _NC + cid
        pltpu.sync_copy(z_hbm, buf_v.at[0])
        for r in range(_RPT // c):
            pltpu.sync_copy(buf_v.at[0], acc_sh.at[pl.ds(sid * _RPT + r * c, c)])
        if _RPT % c:
            pltpu.sync_copy(buf_v.at[0, pl.ds(0, _RPT % c)],
                            acc_sh.at[pl.ds(sid * _RPT + (_RPT // c) * c, _RPT % c)])
        plsc.subcore_barrier()

        def idx_issue(j, s):
            pltpu.async_copy(src_hbm.at[wid, j], srcix.at[s], isem)
            pltpu.async_copy(dst_hbm.at[wid, j], dstix.at[s], isem)

        def idx_wait(j, s):
            pltpu.make_async_copy(src_hbm.at[wid, j], srcix.at[s], isem).wait()
            pltpu.make_async_copy(dst_hbm.at[wid, j], dstix.at[s], isem).wait()

        def gat_issue(s, bs):
            pltpu.async_copy(g_hbm.at[srcix.at[s]], buf_v.at[bs], gsem)

        def gat_wait(s, bs):
            pltpu.make_async_copy(g_hbm.at[srcix.at[s]], buf_v.at[bs], gsem).wait()

        def sca_issue(s, bs):
            pltpu.async_copy(buf_v.at[bs], acc_sh.at[dstix.at[s]], ssem, add=True)

        def sca_wait(s, bs):
            pltpu.make_async_copy(buf_v.at[bs], acc_sh.at[dstix.at[s]], ssem).wait()

        for t in range(nb - 1):
            idx_issue(t, t)
        for t in range(gd):
            idx_wait(t, t)
            gat_issue(t, t)

        def group(o, carry):
            j0 = o * ni
            for u in range(ni):
                j = j0 + u

                @pl.when(j + nb - 1 < nch)
                def _():
                    idx_issue(j + nb - 1, (u + nb - 1) % ni)

                @pl.when(j + gd < nch)
                def _():
                    @pl.when(j + gd - nb >= 0)
                    def _():
                        sca_wait((u + gd + nb) % ni, (u + gd) % nb)

                    idx_wait(j + gd, (u + gd) % ni)
                    gat_issue((u + gd) % ni, (u + gd) % nb)

                gat_wait(u % ni, u % nb)
                sca_issue(u % ni, u % nb)

            return carry

        lax.fori_loop(0, nch // ni, group, 0)
        for t in range(nb):
            kk = nch - nb + t
            sca_wait(kk % ni, kk % nb)
        plsc.subcore_barrier()
        pltpu.sync_copy(acc_sh.at[pl.ds(sid * _RPT, _RPT)],
                        out_hbm.at[cid, pl.ds(sid * _RPT, _RPT)])

    return k(g, srcs, dsts, zrows)


def _sc_degree(dsts, zo):
    mesh = plsc.VectorSubcoreMesh(core_axis_name="c", subcore_axis_name="s")

    @functools.partial(
        pl.kernel,
        out_type=jax.ShapeDtypeStruct((_NC, _NPAD, _DEGW), jnp.float32),
        mesh=mesh,
        scratch_types=[
            pltpu.VMEM((_NCH2, _C2), jnp.int32),
            pltpu.VMEM((_C2, _DEGW), jnp.float32),
            pltpu.VMEM_SHARED((_NPAD, _DEGW), jnp.float32),
        ],
        compiler_params=pltpu.CompilerParams(use_tc_tiling_on_sc=False),
    )
    def k(dst_hbm, zo_hbm, out_hbm, dst_v, val_v, acc_sh):
        cid = lax.axis_index("c")
        sid = lax.axis_index("s")
        wid = sid * _NC + cid
        pltpu.sync_copy(dst_hbm.at[wid], dst_v)
        pltpu.sync_copy(zo_hbm.at[0], val_v)
        for r in range(_RPT // _C2):
            pltpu.sync_copy(val_v, acc_sh.at[pl.ds(sid * _RPT + r * _C2, _C2)])
        plsc.subcore_barrier()
        pltpu.sync_copy(zo_hbm.at[1], val_v)

        def body(j, carry):
            pltpu.sync_copy(val_v, acc_sh.at[dst_v.at[j]], add=True)
            return carry

        lax.fori_loop(0, _NCH2, body, 0)
        plsc.subcore_barrier()
        pltpu.sync_copy(acc_sh.at[pl.ds(sid * _RPT, _RPT)],
                        out_hbm.at[cid, pl.ds(sid * _RPT, _RPT)])

    return k(dsts, zo)


def _dinv(degp_ref):
    deg = degp_ref[0] + degp_ref[1]
    return jnp.where(deg > 0, lax.rsqrt(jnp.maximum(deg, 1e-12)), 0.0)


def _tc1_kernel(degp_ref, x_ref, w1_ref, g_ref):
    dinv = _dinv(degp_ref)
    h = jnp.dot(x_ref[...], w1_ref[...], preferred_element_type=jnp.float32)
    g_ref[...] = dinv[:_N, 0:1] * h


def _tc2_kernel(p1_ref, degp_ref, b1_ref, w2_ref, g2_ref):
    dinv = _dinv(degp_ref)
    agg = p1_ref[0, :_N] + p1_ref[1, :_N]
    z = jnp.maximum(dinv[:_N, 0:1] * agg + b1_ref[...], 0.0)
    h2 = jnp.dot(z, w2_ref[...], preferred_element_type=jnp.float32)
    g2_ref[...] = dinv[:_N, 0:1] * h2


def _tc3_kernel(p2_ref, degp_ref, b2_ref, o_ref):
    dinv = _dinv(degp_ref)
    agg = p2_ref[0, :_N] + p2_ref[1, :_N]
    o = dinv[:_N, 0:1] * agg + b2_ref[...]
    col = lax.broadcasted_iota(jnp.int32, (_N, _D2), 1)
    logits = jnp.where(col < _NCLS, o, -jnp.inf)
    m = jnp.max(logits, axis=1, keepdims=True)
    ex = jnp.where(col < _NCLS, jnp.exp(o - m), 0.0)
    lse = jnp.log(jnp.sum(ex, axis=1, keepdims=True))
    o_ref[...] = o - m - lse


def _edge_layout(src, dst, c, nch):
    epad = _NW * nch * c
    pad_n = epad - src.shape[0]
    s = jnp.concatenate([src, jnp.zeros((pad_n,), jnp.int32)]).reshape(_NW, nch, c)
    t = jnp.concatenate(
        [dst, jnp.full((pad_n,), _NPAD - 1, jnp.int32)]).reshape(_NW, nch, c)
    return s, t


def kernel(x, edge_index, W1, b1, W2, b2):
    loops = jnp.arange(_N, dtype=jnp.int32)
    src = jnp.concatenate([edge_index[0].astype(jnp.int32), loops])
    dst = jnp.concatenate([edge_index[1].astype(jnp.int32), loops])
    srcs1, dsts1 = _edge_layout(src, dst, _C1, _NCH1)
    srcs2, dsts2 = _edge_layout(src, dst, _C2, _NCH2)

    zo = jnp.zeros((2, _C2, _DEGW), jnp.float32).at[1, :, 0].set(1.0)
    z1 = jnp.zeros((_C1, _D1), jnp.float32)
    z2 = jnp.zeros((_C2, _D2), jnp.float32)
    w2p = jnp.pad(W2, ((0, 0), (0, _D2 - _NCLS)))
    b2p = jnp.pad(b2, (0, _D2 - _NCLS))

    degp = _sc_degree(dsts2, zo)

    g1 = pl.pallas_call(
        _tc1_kernel,
        out_shape=jax.ShapeDtypeStruct((_N, _D1), jnp.float32),
    )(degp, x, W1)

    p1 = _sc_agg(g1, srcs1, dsts1, z1, _D1, _C1, _NCH1, 3, 1)

    g2 = pl.pallas_call(
        _tc2_kernel,
        out_shape=jax.ShapeDtypeStruct((_N, _D2), jnp.float32),
    )(p1, degp, b1, w2p)

    p2 = _sc_agg(g2, srcs2, dsts2, z2, _D2, _C2, _NCH2, 3, 1)

    out = pl.pallas_call(
        _tc3_kernel,
        out_shape=jax.ShapeDtypeStruct((_N, _D2), jnp.float32),
    )(p2, degp, b2p)

    return out[:, :_NCLS]
